# baseline (speedup 1.0000x reference)
import jax
import jax.numpy as jnp
from jax import lax
from jax.experimental import pallas as pl
from jax.experimental.pallas import tpu as pltpu
from jax.experimental.pallas import tpu_sc as plsc

B, C, T, D = 8, 64, 1024, 64
QL, NCLS, NCH, NSUB = 256, 1000, 64, 1000
NC, NS = 2, 16
CHUNK = 128
NTJ = T // CHUNK


def _body(x_hbm, ids_hbm, sid_hbm, cond_hbm, qt_hbm, condz_hbm, ch_hbm,
          sub_hbm, out_hbm,
          idsv, sidv, che_all, sub_all, cidx, cese, qt, cest, xbuf, buf5,
          gsem, wsem, xsem):
    w = lax.axis_index("s") * NC + lax.axis_index("c")
    b = w // 4
    c0 = (w % 4) * 16
    iota = jnp.arange(16, dtype=jnp.int32)

    pltpu.sync_copy(ids_hbm, idsv)
    pltpu.sync_copy(sid_hbm, sidv)
    pltpu.sync_copy(qt_hbm, qt)
    pltpu.sync_copy(ch_hbm.at[idsv], che_all)
    pltpu.sync_copy(sub_hbm.at[sidv], sub_all)
    pltpu.sync_copy(cond_hbm.at[b], cidx)
    for j in range(NTJ):
        pltpu.async_copy(condz_hbm.at[cidx.at[j]],
                         cese.at[pl.ds(j * CHUNK, CHUNK)], gsem.at[j & 1])
    se_v = [sub_all[b, pl.ds(16 * k, 16)] for k in range(4)]

    @plsc.parallel_loop(0, NCH, 1, unroll=4)
    def fold_se(r):
        for k in range(4):
            plsc.addupdate(che_all.at[r, pl.ds(16 * k, 16)], se_v[k])

    for j in range(NTJ):
        pltpu.make_async_copy(condz_hbm.at[cidx.at[j]],
                              cese.at[pl.ds(j * CHUNK, CHUNK)],
                              gsem.at[j & 1]).wait()

    def write(i, tj):
        return pltpu.make_async_copy(
            buf5.at[i & 1], out_hbm.at[b * C + c0 + i, :, tj], wsem.at[i & 1])

    def xload(j, xq):
        return pltpu.make_async_copy(
            x_hbm.at[b, pl.ds(c0, 16), pl.ds(j * CHUNK, CHUNK)],
            xbuf.at[xq], xsem)

    xload(0, 0).start()
    xload(0, 0).wait()

    def tj_body(tj, carry):
        t0 = tj * CHUNK
        xq = tj & 1

        @pl.when(tj < NTJ - 1)
        def _():
            xload(tj + 1, 1 - xq).start()

        tv = [t0 + 16 * j + iota for j in range(NTJ)]

        @plsc.parallel_loop(0, D, 1, unroll=2)
        def t_body(d):
            dspl = jnp.full((16,), d, jnp.int32)
            for j in range(NTJ):
                cest[d, pl.ds(16 * j, 16)] = plsc.load_gather(
                    cese, [tv[j], dspl])

        def c_body(i, carry2):
            c = c0 + i
            p = i & 1
            xv = [xbuf[xq, i, pl.ds(16 * j, 16)] for j in range(NTJ)]

            @pl.when(i >= 2)
            def _():
                write(i - 2, tj).wait()

            @plsc.parallel_loop(0, D, 1, unroll=4)
            def d_body(d):
                dspl = jnp.full((16,), d, jnp.int32)
                addv = plsc.load_gather(che_all, [jnp.full((16,), c,
                                                          jnp.int32), dspl])
                for j in range(NTJ):
                    qv = plsc.load_gather(qt.at[d], [xv[j]])
                    cv = cest[d, pl.ds(16 * j, 16)]
                    buf5[p, d >> 3, d & 7, pl.ds(16 * j, 16)] = (
                        qv + cv + addv)

            write(i, tj).start()
            return carry2

        lax.fori_loop(0, 16, c_body, 0)
        write(14, tj).wait()
        write(15, tj).wait()

        @pl.when(tj < NTJ - 1)
        def _():
            xload(tj + 1, 1 - xq).wait()
        return carry

    lax.fori_loop(0, NTJ, tj_body, 0)


def kernel(x, ids, cond, sid, quant_table, cond_table, ch_table, sub_table):
    x32 = x.astype(jnp.int32)
    ids32 = ids.astype(jnp.int32)
    cond32 = cond.reshape(B, NTJ, CHUNK).astype(jnp.int32)
    sid32 = jnp.pad(sid.reshape(B).astype(jnp.int32), (0, 8))
    condz = cond_table.at[0].set(0.0)
    qt_t = quant_table.T

    p = pl.kernel(
        _body,
        out_type=jax.ShapeDtypeStruct((B * C, D // 8, NTJ, 8, CHUNK),
                                      jnp.float32),
        mesh=plsc.VectorSubcoreMesh(core_axis_name="c", subcore_axis_name="s",
                                    num_cores=NC, num_subcores=NS),
        compiler_params=pltpu.CompilerParams(use_tc_tiling_on_sc=False,
                                             needs_layout_passes=False),
        scratch_types=[
            pltpu.VMEM((NCH,), jnp.int32),
            pltpu.VMEM((16,), jnp.int32),
            pltpu.VMEM((NCH, D), jnp.float32),
            pltpu.VMEM((16, D), jnp.float32),
            pltpu.VMEM((NTJ, CHUNK), jnp.int32),
            pltpu.VMEM((T, D), jnp.float32),
            pltpu.VMEM((D, QL), jnp.float32),
            pltpu.VMEM((D, CHUNK), jnp.float32),
            pltpu.VMEM((2, 16, CHUNK), jnp.int32),
            pltpu.VMEM((2, 8, 8, CHUNK), jnp.float32),
            pltpu.SemaphoreType.DMA((2,)),
            pltpu.SemaphoreType.DMA((2,)),
            pltpu.SemaphoreType.DMA,
        ],
    )
    o5 = p(x32, ids32, sid32, cond32, qt_t, condz, ch_table, sub_table)
    return o5.transpose(0, 2, 4, 1, 3).reshape(B * C, T, D)


# speedup vs baseline: 1.0259x; 1.0259x over previous
import jax
import jax.numpy as jnp
from jax import lax
from jax.experimental import pallas as pl
from jax.experimental.pallas import tpu as pltpu
from jax.experimental.pallas import tpu_sc as plsc

B, C, T, D = 8, 64, 1024, 64
QL, NCLS, NCH, NSUB = 256, 1000, 64, 1000
NC, NS = 2, 16
CHUNK = 128
NTJ = T // CHUNK


def _body(x_hbm, ids_hbm, sid_hbm, cond_hbm, qt_hbm, condz_hbm, ch_hbm,
          sub_hbm, out_hbm,
          idsv, sidv, che_all, sub_all, cidx, cese, qt, cest, xbuf, buf5,
          gsem, wsem, xsem):
    w = lax.axis_index("s") * NC + lax.axis_index("c")
    b = w // 4
    c0 = (w % 4) * 16
    iota = jnp.arange(16, dtype=jnp.int32)

    pltpu.sync_copy(ids_hbm, idsv)
    pltpu.sync_copy(sid_hbm, sidv)
    pltpu.sync_copy(qt_hbm, qt)
    pltpu.sync_copy(ch_hbm.at[idsv], che_all)
    pltpu.sync_copy(sub_hbm.at[sidv], sub_all)
    pltpu.sync_copy(cond_hbm.at[b], cidx)
    for j in range(NTJ):
        pltpu.async_copy(condz_hbm.at[cidx.at[j]],
                         cese.at[pl.ds(j * CHUNK, CHUNK)], gsem.at[j & 1])
    se_v = [sub_all[b, pl.ds(16 * k, 16)] for k in range(4)]

    @plsc.parallel_loop(0, NCH, 1, unroll=4)
    def fold_se(r):
        for k in range(4):
            plsc.addupdate(che_all.at[r, pl.ds(16 * k, 16)], se_v[k])

    for j in range(NTJ):
        pltpu.make_async_copy(condz_hbm.at[cidx.at[j]],
                              cese.at[pl.ds(j * CHUNK, CHUNK)],
                              gsem.at[j & 1]).wait()

    def write(i, tj):
        return pltpu.make_async_copy(
            buf5.at[i & 1], out_hbm.at[b * C + c0 + i, :, tj], wsem.at[i & 1])

    def xload(j, xq):
        return pltpu.make_async_copy(
            x_hbm.at[b, pl.ds(c0, 16), pl.ds(j * CHUNK, CHUNK)],
            xbuf.at[xq], xsem)

    xload(0, 0).start()
    xload(0, 0).wait()

    def tj_body(tj, carry):
        t0 = tj * CHUNK
        xq = tj & 1

        @pl.when(tj < NTJ - 1)
        def _():
            xload(tj + 1, 1 - xq).start()

        tv = [t0 + 16 * j + iota for j in range(NTJ)]

        @plsc.parallel_loop(0, D, 1, unroll=2)
        def t_body(d):
            dspl = jnp.full((16,), d, jnp.int32)
            for j in range(NTJ):
                cest[d, pl.ds(16 * j, 16)] = plsc.load_gather(
                    cese, [tv[j], dspl])

        def c_body(i, carry2):
            c = c0 + i
            p = i & 1
            xv = [xbuf[xq, i, pl.ds(16 * j, 16)] for j in range(NTJ)]

            @pl.when(i >= 2)
            def _():
                write(i - 2, tj).wait()

            @plsc.parallel_loop(0, D, 1, unroll=2)
            def d_body(d):
                dspl = jnp.full((16,), d, jnp.int32)
                addv = plsc.load_gather(che_all, [jnp.full((16,), c,
                                                          jnp.int32), dspl])
                for j in range(NTJ):
                    qv = plsc.load_gather(qt.at[d], [xv[j]])
                    cv = cest[d, pl.ds(16 * j, 16)]
                    buf5[p, d >> 3, d & 7, pl.ds(16 * j, 16)] = (
                        qv + cv + addv)

            write(i, tj).start()
            return carry2

        lax.fori_loop(0, 16, c_body, 0)
        write(14, tj).wait()
        write(15, tj).wait()

        @pl.when(tj < NTJ - 1)
        def _():
            xload(tj + 1, 1 - xq).wait()
        return carry

    lax.fori_loop(0, NTJ, tj_body, 0)


def kernel(x, ids, cond, sid, quant_table, cond_table, ch_table, sub_table):
    x32 = x.astype(jnp.int32)
    ids32 = ids.astype(jnp.int32)
    cond32 = cond.reshape(B, NTJ, CHUNK).astype(jnp.int32)
    sid32 = jnp.pad(sid.reshape(B).astype(jnp.int32), (0, 8))
    condz = cond_table.at[0].set(0.0)
    qt_t = quant_table.T

    p = pl.kernel(
        _body,
        out_type=jax.ShapeDtypeStruct((B * C, D // 8, NTJ, 8, CHUNK),
                                      jnp.float32),
        mesh=plsc.VectorSubcoreMesh(core_axis_name="c", subcore_axis_name="s",
                                    num_cores=NC, num_subcores=NS),
        compiler_params=pltpu.CompilerParams(use_tc_tiling_on_sc=False,
                                             needs_layout_passes=False),
        scratch_types=[
            pltpu.VMEM((NCH,), jnp.int32),
            pltpu.VMEM((16,), jnp.int32),
            pltpu.VMEM((NCH, D), jnp.float32),
            pltpu.VMEM((16, D), jnp.float32),
            pltpu.VMEM((NTJ, CHUNK), jnp.int32),
            pltpu.VMEM((T, D), jnp.float32),
            pltpu.VMEM((D, QL), jnp.float32),
            pltpu.VMEM((D, CHUNK), jnp.float32),
            pltpu.VMEM((2, 16, CHUNK), jnp.int32),
            pltpu.VMEM((2, 8, 8, CHUNK), jnp.float32),
            pltpu.SemaphoreType.DMA((2,)),
            pltpu.SemaphoreType.DMA((2,)),
            pltpu.SemaphoreType.DMA,
        ],
    )
    o5 = p(x32, ids32, sid32, cond32, qt_t, condz, ch_table, sub_table)
    return o5.transpose(0, 2, 4, 1, 3).reshape(B * C, T, D)
